# baseline (speedup 1.0000x reference)
import jax
import jax.numpy as jnp
from jax.experimental import pallas as pl


def _euler_to_rot(euler):
    x, y, z = euler[:, 0], euler[:, 1], euler[:, 2]
    cx_, sx_ = jnp.cos(x), jnp.sin(x)
    cy_, sy_ = jnp.cos(y), jnp.sin(y)
    cz_, sz_ = jnp.cos(z), jnp.sin(z)
    zeros = jnp.zeros_like(x); ones = jnp.ones_like(x)
    Rx = jnp.stack([ones, zeros, zeros, zeros, cx_, -sx_, zeros, sx_, cx_], -1).reshape(-1, 3, 3)
    Ry = jnp.stack([cy_, zeros, sy_, zeros, ones, zeros, -sy_, zeros, cy_], -1).reshape(-1, 3, 3)
    Rz = jnp.stack([cz_, -sz_, zeros, sz_, cz_, zeros, zeros, zeros, ones], -1).reshape(-1, 3, 3)
    return Rz @ Ry @ Rx


def _copy_body(x_ref, o_ref):
    o_ref[...] = x_ref[...]


def kernel(geometry, euler, trans, cam, tris, vert_tris):
    R = _euler_to_rot(euler)
    rott_geo = jnp.einsum('bij,bnj->bni', R, geometry) + trans[:, None, :]
    fx, fy, ux, uy = cam[:, 0], cam[:, 1], cam[:, 2], cam[:, 3]
    z = rott_geo[..., 2]
    proj_geo = jnp.stack([
        rott_geo[..., 0] / z * fx[:, None] + ux[:, None],
        rott_geo[..., 1] / z * fy[:, None] + uy[:, None],
        z,
    ], axis=-1)
    vert_1 = jnp.take(rott_geo, tris[:, 0], axis=1)
    vert_2 = jnp.take(rott_geo, tris[:, 1], axis=1)
    vert_3 = jnp.take(rott_geo, tris[:, 2], axis=1)
    nnorm = jnp.cross(vert_2 - vert_1, vert_3 - vert_1)
    rot_tri_normal = nnorm / jnp.maximum(jnp.linalg.norm(nnorm, axis=-1, keepdims=True), 1e-12)
    rot_vert_normal = jnp.take(rot_tri_normal, vert_tris, axis=1)
    view_dir = rott_geo / jnp.maximum(jnp.linalg.norm(rott_geo, axis=-1, keepdims=True), 1e-12)
    is_visible = -jnp.sum(rot_vert_normal * view_dir, axis=-1)
    is_visible = jnp.where(is_visible < 0.01, -1.0, is_visible)
    is_visible = pl.pallas_call(
        _copy_body,
        out_shape=jax.ShapeDtypeStruct(is_visible.shape, is_visible.dtype),
    )(is_visible)
    return (proj_geo, rot_tri_normal, is_visible)


# speedup vs baseline: 3.5310x; 3.5310x over previous
import functools

import jax
import jax.numpy as jnp
from jax import lax
from jax.experimental import pallas as pl
from jax.experimental.pallas import tpu as pltpu
from jax.experimental.pallas import tpu_sc as plsc

B, N, T = 4, 100000, 200000
NC, NS, L = 2, 16, 16
NW = NC * NS
C = 112
NG = C // L
VPW = 3136
TPW = 6272
NCH_V = VPW // C
NCH_T = TPW // C

_f32 = jnp.float32
_i32 = jnp.int32


def _euler_rotmats(euler):
    x, y, z = euler[:, 0], euler[:, 1], euler[:, 2]
    cx_, sx_ = jnp.cos(x), jnp.sin(x)
    cy_, sy_ = jnp.cos(y), jnp.sin(y)
    cz_, sz_ = jnp.cos(z), jnp.sin(z)
    zeros = jnp.zeros_like(x); ones = jnp.ones_like(x)
    Rx = jnp.stack([ones, zeros, zeros, zeros, cx_, -sx_, zeros, sx_, cx_], -1).reshape(-1, 3, 3)
    Ry = jnp.stack([cy_, zeros, sy_, zeros, ones, zeros, -sy_, zeros, cy_], -1).reshape(-1, 3, 3)
    Rz = jnp.stack([cz_, -sz_, zeros, sz_, cz_, zeros, zeros, zeros, ones], -1).reshape(-1, 3, 3)
    return Rx, Ry, Rz


def _fl(k):
    return jnp.full((L,), k, _i32)


def _iota():
    return lax.iota(_i32, L)


def _inv_len(ss):
    i = plsc.bitcast(ss, _i32)
    i = jnp.int32(0x5F3759DF) - lax.shift_right_arithmetic(i, 1)
    y = plsc.bitcast(i, _f32)
    h = ss * _f32(0.5)
    for _ in range(3):
        y = y * (_f32(1.5) - (h * y) * y)
    return jnp.where(ss < _f32(1e-24), _f32(1e12), y)


def _bf16_round(v):
    i = plsc.bitcast(v, _i32)
    tie = jnp.bitwise_and(lax.shift_right_arithmetic(i, 16), jnp.int32(1))
    i = i + jnp.int32(0x7FFF) + tie
    i = jnp.bitwise_and(i, jnp.int32(-65536))
    return plsc.bitcast(i, _f32)


def _sum3_rne(p0, p1, p2):
    t = p0 + p1
    bv = t - p0
    e1 = (p0 - (t - bv)) + (p1 - bv)
    s = t + p2
    bv2 = s - t
    e2 = (t - (s - bv2)) + (p2 - bv2)
    return s + (e1 + e2)


def _worker_id():
    return lax.axis_index("s") * NC + lax.axis_index("c")


def _k1_body(geom, params, proj, packed, gbuf, pbuf, projbuf, parbuf):
    wid = _worker_id()
    pltpu.sync_copy(params, parbuf)
    iota = _iota()

    def chunk(j, carry):
        base = jnp.minimum(wid * VPW + j * C, N - C)
        for b in range(B):
            pltpu.sync_copy(geom.at[b, pl.ds(base, C), :], gbuf.at[b])
        for g in range(NG):
            row = g * L + iota
            for b in range(B):
                gx = _bf16_round(plsc.load_gather(gbuf, [_fl(b), row, _fl(0)]))
                gy = _bf16_round(plsc.load_gather(gbuf, [_fl(b), row, _fl(1)]))
                gz = _bf16_round(plsc.load_gather(gbuf, [_fl(b), row, _fl(2)]))
                P = lambda k: parbuf[b, k]
                rx = _sum3_rne(P(0) * gx, P(1) * gy, P(2) * gz) + P(9)
                ry = _sum3_rne(P(3) * gx, P(4) * gy, P(5) * gz) + P(10)
                rz = _sum3_rne(P(6) * gx, P(7) * gy, P(8) * gz) + P(11)
                px = rx / rz * P(12) + P(14)
                py = ry / rz * P(13) + P(15)
                a = 3 * b
                plsc.store_scatter(pbuf, [row, _fl(a + 0)], rx)
                plsc.store_scatter(pbuf, [row, _fl(a + 1)], ry)
                plsc.store_scatter(pbuf, [row, _fl(a + 2)], rz)
                plsc.store_scatter(projbuf, [_fl(b), row, _fl(0)], px)
                plsc.store_scatter(projbuf, [_fl(b), row, _fl(1)], py)
                plsc.store_scatter(projbuf, [_fl(b), row, _fl(2)], rz)
        pltpu.sync_copy(pbuf, packed.at[pl.ds(base, C), :])
        for b in range(B):
            pltpu.sync_copy(projbuf.at[b], proj.at[b, pl.ds(base, C), :])
        return carry

    lax.fori_loop(0, NCH_V, chunk, 0)


def _k2_body(tris4, packed, tri_out, npacked,
             tbuf, i0, i1, i2, v0, v1, v2, nbuf, aosbuf, sem):
    wid = _worker_id()
    iota = _iota()

    def chunk(j, carry):
        base = jnp.minimum(wid * TPW + j * C, T - C)
        pltpu.sync_copy(tris4.at[pl.ds(base, C), :], tbuf)
        for g in range(NG):
            row = g * L + iota
            plsc.store_scatter(i0, [row], plsc.load_gather(tbuf, [row, _fl(0)]))
            plsc.store_scatter(i1, [row], plsc.load_gather(tbuf, [row, _fl(1)]))
            plsc.store_scatter(i2, [row], plsc.load_gather(tbuf, [row, _fl(2)]))
        cp0 = pltpu.async_copy(packed.at[i0], v0, sem)
        cp1 = pltpu.async_copy(packed.at[i1], v1, sem)
        cp2 = pltpu.async_copy(packed.at[i2], v2, sem)
        cp0.wait(); cp1.wait(); cp2.wait()
        for g in range(NG):
            row = g * L + iota
            for b in range(B):
                a = 3 * b
                x1 = plsc.load_gather(v0, [row, _fl(a + 0)])
                y1 = plsc.load_gather(v0, [row, _fl(a + 1)])
                z1 = plsc.load_gather(v0, [row, _fl(a + 2)])
                x2 = plsc.load_gather(v1, [row, _fl(a + 0)])
                y2 = plsc.load_gather(v1, [row, _fl(a + 1)])
                z2 = plsc.load_gather(v1, [row, _fl(a + 2)])
                x3 = plsc.load_gather(v2, [row, _fl(a + 0)])
                y3 = plsc.load_gather(v2, [row, _fl(a + 1)])
                z3 = plsc.load_gather(v2, [row, _fl(a + 2)])
                e1x = x2 - x1; e1y = y2 - y1; e1z = z2 - z1
                e2x = x3 - x1; e2y = y3 - y1; e2z = z3 - z1
                cx = e1y * e2z - e1z * e2y
                cy = e1z * e2x - e1x * e2z
                cz = e1x * e2y - e1y * e2x
                s = _inv_len(cx * cx + cy * cy + cz * cz)
                nx = cx * s; ny = cy * s; nz = cz * s
                plsc.store_scatter(nbuf, [row, _fl(a + 0)], nx)
                plsc.store_scatter(nbuf, [row, _fl(a + 1)], ny)
                plsc.store_scatter(nbuf, [row, _fl(a + 2)], nz)
                plsc.store_scatter(aosbuf, [_fl(b), row, _fl(0)], nx)
                plsc.store_scatter(aosbuf, [_fl(b), row, _fl(1)], ny)
                plsc.store_scatter(aosbuf, [_fl(b), row, _fl(2)], nz)
        pltpu.sync_copy(nbuf, npacked.at[pl.ds(base, C), :])
        for b in range(B):
            pltpu.sync_copy(aosbuf.at[b], tri_out.at[b, pl.ds(base, C), :])
        return carry

    lax.fori_loop(0, NCH_T, chunk, 0)


def _k3_body(vt, npacked, packed, vis_out, ibuf, nbuf, rbuf, vbuf, sem):
    wid = _worker_id()
    iota = _iota()

    def chunk(j, carry):
        base = jnp.minimum(wid * VPW + j * C, N - C)
        pltpu.sync_copy(vt.at[pl.ds(base, C)], ibuf)
        pltpu.async_copy(npacked.at[ibuf], nbuf, sem).wait()
        pltpu.sync_copy(packed.at[pl.ds(base, C), :], rbuf)
        for g in range(NG):
            row = g * L + iota
            for b in range(B):
                a = 3 * b
                nx = plsc.load_gather(nbuf, [row, _fl(a + 0)])
                ny = plsc.load_gather(nbuf, [row, _fl(a + 1)])
                nz = plsc.load_gather(nbuf, [row, _fl(a + 2)])
                rx = plsc.load_gather(rbuf, [row, _fl(a + 0)])
                ry = plsc.load_gather(rbuf, [row, _fl(a + 1)])
                rz = plsc.load_gather(rbuf, [row, _fl(a + 2)])
                s = _inv_len(rx * rx + ry * ry + rz * rz)
                vis = -(nx * rx + ny * ry + nz * rz) * s
                vis = jnp.where(vis < _f32(0.01), _f32(-1.0), vis)
                plsc.store_scatter(vbuf, [_fl(b), row], vis)
        for b in range(B):
            pltpu.sync_copy(vbuf.at[b], vis_out.at[b, pl.ds(base, C)])
        return carry

    lax.fori_loop(0, NCH_V, chunk, 0)


_CPARAMS = pltpu.CompilerParams(needs_layout_passes=False,
                                use_tc_tiling_on_sc=False)
_KERNELS = {}


def _build_kernels():
    if _KERNELS:
        return _KERNELS
    mesh = plsc.VectorSubcoreMesh(core_axis_name="c", subcore_axis_name="s",
                                  num_cores=NC, num_subcores=NS)
    _KERNELS["k1"] = pl.kernel(
        _k1_body,
        compiler_params=_CPARAMS,
        out_type=[jax.ShapeDtypeStruct((B, N, 3), _f32),
                  jax.ShapeDtypeStruct((N, 16), _f32)],
        mesh=mesh,
        scratch_types=[pltpu.VMEM((B, C, 4), _f32),
                       pltpu.VMEM((C, 16), _f32),
                       pltpu.VMEM((B, C, 3), _f32),
                       pltpu.VMEM((B, 16, L), _f32)],
    )
    _KERNELS["k2"] = pl.kernel(
        _k2_body,
        compiler_params=_CPARAMS,
        out_type=[jax.ShapeDtypeStruct((B, T, 3), _f32),
                  jax.ShapeDtypeStruct((T, 16), _f32)],
        mesh=mesh,
        scratch_types=[pltpu.VMEM((C, 4), _i32),
                       pltpu.VMEM((C,), _i32),
                       pltpu.VMEM((C,), _i32),
                       pltpu.VMEM((C,), _i32),
                       pltpu.VMEM((C, 16), _f32),
                       pltpu.VMEM((C, 16), _f32),
                       pltpu.VMEM((C, 16), _f32),
                       pltpu.VMEM((C, 16), _f32),
                       pltpu.VMEM((B, C, 3), _f32),
                       pltpu.SemaphoreType.DMA],
    )
    _KERNELS["k3"] = pl.kernel(
        _k3_body,
        compiler_params=_CPARAMS,
        out_type=jax.ShapeDtypeStruct((B, N), _f32),
        mesh=mesh,
        scratch_types=[pltpu.VMEM((C,), _i32),
                       pltpu.VMEM((C, 16), _f32),
                       pltpu.VMEM((C, 16), _f32),
                       pltpu.VMEM((B, C), _f32),
                       pltpu.SemaphoreType.DMA],
    )
    return _KERNELS


def _bf16r(x):
    i = lax.bitcast_convert_type(x, jnp.int32)
    tie = jnp.bitwise_and(lax.shift_right_logical(i, 16), jnp.int32(1))
    i = i + jnp.int32(0x7FFF) + tie
    i = jnp.bitwise_and(i, jnp.int32(-65536))
    return lax.bitcast_convert_type(i, jnp.float32)


def _sum3_rne_jnp(p0, p1, p2):
    t = p0 + p1
    bv = t - p0
    e1 = (p0 - (t - bv)) + (p1 - bv)
    s = t + p2
    bv2 = s - t
    e2 = (t - (s - bv2)) + (p2 - bv2)
    return s + (e1 + e2)


def _mm3_bf16x1(A, Bm):
    Ab, Bb = _bf16r(A), _bf16r(Bm)
    p0 = Ab[:, :, 0, None] * Bb[:, 0, None, :].reshape(-1, 1, 3)
    p1 = Ab[:, :, 1, None] * Bb[:, 1, None, :].reshape(-1, 1, 3)
    p2 = Ab[:, :, 2, None] * Bb[:, 2, None, :].reshape(-1, 1, 3)
    return _sum3_rne_jnp(p0, p1, p2)


def kernel(geometry, euler, trans, cam, tris, vert_tris):
    Rx, Ry, Rz = _euler_rotmats(euler)
    R = _bf16r(_mm3_bf16x1(_mm3_bf16x1(Rz, Ry), Rx))
    params = jnp.concatenate([R.reshape(B, 9), trans, cam], axis=1)
    params_bc = jnp.broadcast_to(params[:, :, None], (B, 16, L)).astype(_f32)
    geom4 = jnp.concatenate(
        [geometry, jnp.zeros((B, N, 1), _f32)], axis=-1)
    tris4 = jnp.concatenate(
        [tris, jnp.zeros((T, 1), _i32)], axis=-1)

    ks = _build_kernels()
    proj_geo, packed = ks["k1"](geom4, params_bc)
    rot_tri_normal, npacked = ks["k2"](tris4, packed)
    is_visible = ks["k3"](vert_tris, npacked, packed)
    return (proj_geo, rot_tri_normal, is_visible)


# speedup vs baseline: 3.5328x; 1.0005x over previous
import functools

import jax
import jax.numpy as jnp
from jax import lax
from jax.experimental import pallas as pl
from jax.experimental.pallas import tpu as pltpu
from jax.experimental.pallas import tpu_sc as plsc

B, N, T = 4, 100000, 200000
NC, NS, L = 2, 16, 16
NW = NC * NS
C = 112
NG = C // L
VPW = 3136
TPW = 6272
NCH_V = VPW // C
NCH_T = TPW // C

_f32 = jnp.float32
_i32 = jnp.int32


def _euler_rotmats(euler):
    x, y, z = euler[:, 0], euler[:, 1], euler[:, 2]
    cx_, sx_ = jnp.cos(x), jnp.sin(x)
    cy_, sy_ = jnp.cos(y), jnp.sin(y)
    cz_, sz_ = jnp.cos(z), jnp.sin(z)
    zeros = jnp.zeros_like(x); ones = jnp.ones_like(x)
    Rx = jnp.stack([ones, zeros, zeros, zeros, cx_, -sx_, zeros, sx_, cx_], -1).reshape(-1, 3, 3)
    Ry = jnp.stack([cy_, zeros, sy_, zeros, ones, zeros, -sy_, zeros, cy_], -1).reshape(-1, 3, 3)
    Rz = jnp.stack([cz_, -sz_, zeros, sz_, cz_, zeros, zeros, zeros, ones], -1).reshape(-1, 3, 3)
    return Rx, Ry, Rz


def _fl(k):
    return jnp.full((L,), k, _i32)


def _iota():
    return lax.iota(_i32, L)


def _inv_len(ss):
    i = plsc.bitcast(ss, _i32)
    i = jnp.int32(0x5F3759DF) - lax.shift_right_arithmetic(i, 1)
    y = plsc.bitcast(i, _f32)
    h = ss * _f32(0.5)
    for _ in range(3):
        y = y * (_f32(1.5) - (h * y) * y)
    return jnp.where(ss < _f32(1e-24), _f32(1e12), y)


def _bf16_round(v):
    i = plsc.bitcast(v, _i32)
    tie = jnp.bitwise_and(lax.shift_right_arithmetic(i, 16), jnp.int32(1))
    i = i + jnp.int32(0x7FFF) + tie
    i = jnp.bitwise_and(i, jnp.int32(-65536))
    return plsc.bitcast(i, _f32)


def _sum3_rne(p0, p1, p2):
    t = p0 + p1
    bv = t - p0
    e1 = (p0 - (t - bv)) + (p1 - bv)
    s = t + p2
    bv2 = s - t
    e2 = (t - (s - bv2)) + (p2 - bv2)
    return s + (e1 + e2)


def _worker_id():
    return lax.axis_index("s") * NC + lax.axis_index("c")


def _k1_body(geom, params, proj, packed, gbuf, pbuf, projbuf, parbuf):
    wid = _worker_id()
    pltpu.sync_copy(params, parbuf)
    iota = _iota()

    def chunk(j, carry):
        base = jnp.minimum(wid * VPW + j * C, N - C)
        for b in range(B):
            pltpu.sync_copy(geom.at[b, pl.ds(base, C), :], gbuf.at[b])
        for g in range(NG):
            row = g * L + iota
            for b in range(B):
                gx = _bf16_round(plsc.load_gather(gbuf, [_fl(b), row, _fl(0)]))
                gy = _bf16_round(plsc.load_gather(gbuf, [_fl(b), row, _fl(1)]))
                gz = _bf16_round(plsc.load_gather(gbuf, [_fl(b), row, _fl(2)]))
                P = lambda k: parbuf[b, k]
                rx = _sum3_rne(P(0) * gx, P(1) * gy, P(2) * gz) + P(9)
                ry = _sum3_rne(P(3) * gx, P(4) * gy, P(5) * gz) + P(10)
                rz = _sum3_rne(P(6) * gx, P(7) * gy, P(8) * gz) + P(11)
                px = rx / rz * P(12) + P(14)
                py = ry / rz * P(13) + P(15)
                a = 3 * b
                plsc.store_scatter(pbuf, [row, _fl(a + 0)], rx)
                plsc.store_scatter(pbuf, [row, _fl(a + 1)], ry)
                plsc.store_scatter(pbuf, [row, _fl(a + 2)], rz)
                plsc.store_scatter(projbuf, [_fl(b), row, _fl(0)], px)
                plsc.store_scatter(projbuf, [_fl(b), row, _fl(1)], py)
                plsc.store_scatter(projbuf, [_fl(b), row, _fl(2)], rz)
        pltpu.sync_copy(pbuf, packed.at[pl.ds(base, C), :])
        for b in range(B):
            pltpu.sync_copy(projbuf.at[b], proj.at[b, pl.ds(base, C), :])
        return carry

    lax.fori_loop(0, NCH_V, chunk, 0)


def _k2_body(tris3, packed, tri_out, npacked,
             tbuf, i0, i1, i2, v0, v1, v2, nbuf, aosbuf, sem):
    wid = _worker_id()
    iota = _iota()

    def chunk(j, carry):
        base = jnp.minimum(wid * TPW + j * C, T - C)
        pltpu.sync_copy(tris3.at[pl.ds(base, C), :], tbuf)
        for g in range(NG):
            row = g * L + iota
            plsc.store_scatter(i0, [row], plsc.load_gather(tbuf, [row, _fl(0)]))
            plsc.store_scatter(i1, [row], plsc.load_gather(tbuf, [row, _fl(1)]))
            plsc.store_scatter(i2, [row], plsc.load_gather(tbuf, [row, _fl(2)]))
        cp0 = pltpu.async_copy(packed.at[i0], v0, sem)
        cp1 = pltpu.async_copy(packed.at[i1], v1, sem)
        cp2 = pltpu.async_copy(packed.at[i2], v2, sem)
        cp0.wait(); cp1.wait(); cp2.wait()
        for g in range(NG):
            row = g * L + iota
            for b in range(B):
                a = 3 * b
                x1 = plsc.load_gather(v0, [row, _fl(a + 0)])
                y1 = plsc.load_gather(v0, [row, _fl(a + 1)])
                z1 = plsc.load_gather(v0, [row, _fl(a + 2)])
                x2 = plsc.load_gather(v1, [row, _fl(a + 0)])
                y2 = plsc.load_gather(v1, [row, _fl(a + 1)])
                z2 = plsc.load_gather(v1, [row, _fl(a + 2)])
                x3 = plsc.load_gather(v2, [row, _fl(a + 0)])
                y3 = plsc.load_gather(v2, [row, _fl(a + 1)])
                z3 = plsc.load_gather(v2, [row, _fl(a + 2)])
                e1x = x2 - x1; e1y = y2 - y1; e1z = z2 - z1
                e2x = x3 - x1; e2y = y3 - y1; e2z = z3 - z1
                cx = e1y * e2z - e1z * e2y
                cy = e1z * e2x - e1x * e2z
                cz = e1x * e2y - e1y * e2x
                s = _inv_len(cx * cx + cy * cy + cz * cz)
                nx = cx * s; ny = cy * s; nz = cz * s
                plsc.store_scatter(nbuf, [row, _fl(a + 0)], nx)
                plsc.store_scatter(nbuf, [row, _fl(a + 1)], ny)
                plsc.store_scatter(nbuf, [row, _fl(a + 2)], nz)
                plsc.store_scatter(aosbuf, [_fl(b), row, _fl(0)], nx)
                plsc.store_scatter(aosbuf, [_fl(b), row, _fl(1)], ny)
                plsc.store_scatter(aosbuf, [_fl(b), row, _fl(2)], nz)
        pltpu.sync_copy(nbuf, npacked.at[pl.ds(base, C), :])
        for b in range(B):
            pltpu.sync_copy(aosbuf.at[b], tri_out.at[b, pl.ds(base, C), :])
        return carry

    lax.fori_loop(0, NCH_T, chunk, 0)


def _k3_body(vt, npacked, packed, vis_out, ibuf, nbuf, rbuf, vbuf, sem):
    wid = _worker_id()
    iota = _iota()

    def chunk(j, carry):
        base = jnp.minimum(wid * VPW + j * C, N - C)
        pltpu.sync_copy(vt.at[pl.ds(base, C)], ibuf)
        pltpu.async_copy(npacked.at[ibuf], nbuf, sem).wait()
        pltpu.sync_copy(packed.at[pl.ds(base, C), :], rbuf)
        for g in range(NG):
            row = g * L + iota
            for b in range(B):
                a = 3 * b
                nx = plsc.load_gather(nbuf, [row, _fl(a + 0)])
                ny = plsc.load_gather(nbuf, [row, _fl(a + 1)])
                nz = plsc.load_gather(nbuf, [row, _fl(a + 2)])
                rx = plsc.load_gather(rbuf, [row, _fl(a + 0)])
                ry = plsc.load_gather(rbuf, [row, _fl(a + 1)])
                rz = plsc.load_gather(rbuf, [row, _fl(a + 2)])
                s = _inv_len(rx * rx + ry * ry + rz * rz)
                vis = -(nx * rx + ny * ry + nz * rz) * s
                vis = jnp.where(vis < _f32(0.01), _f32(-1.0), vis)
                plsc.store_scatter(vbuf, [_fl(b), row], vis)
        for b in range(B):
            pltpu.sync_copy(vbuf.at[b], vis_out.at[b, pl.ds(base, C)])
        return carry

    lax.fori_loop(0, NCH_V, chunk, 0)


_CPARAMS = pltpu.CompilerParams(needs_layout_passes=False,
                                use_tc_tiling_on_sc=False)
_KERNELS = {}


def _build_kernels():
    if _KERNELS:
        return _KERNELS
    mesh = plsc.VectorSubcoreMesh(core_axis_name="c", subcore_axis_name="s",
                                  num_cores=NC, num_subcores=NS)
    _KERNELS["k1"] = pl.kernel(
        _k1_body,
        compiler_params=_CPARAMS,
        out_type=[jax.ShapeDtypeStruct((B, N, 3), _f32),
                  jax.ShapeDtypeStruct((N, 16), _f32)],
        mesh=mesh,
        scratch_types=[pltpu.VMEM((B, C, 3), _f32),
                       pltpu.VMEM((C, 16), _f32),
                       pltpu.VMEM((B, C, 3), _f32),
                       pltpu.VMEM((B, 16, L), _f32)],
    )
    _KERNELS["k2"] = pl.kernel(
        _k2_body,
        compiler_params=_CPARAMS,
        out_type=[jax.ShapeDtypeStruct((B, T, 3), _f32),
                  jax.ShapeDtypeStruct((T, 16), _f32)],
        mesh=mesh,
        scratch_types=[pltpu.VMEM((C, 3), _i32),
                       pltpu.VMEM((C,), _i32),
                       pltpu.VMEM((C,), _i32),
                       pltpu.VMEM((C,), _i32),
                       pltpu.VMEM((C, 16), _f32),
                       pltpu.VMEM((C, 16), _f32),
                       pltpu.VMEM((C, 16), _f32),
                       pltpu.VMEM((C, 16), _f32),
                       pltpu.VMEM((B, C, 3), _f32),
                       pltpu.SemaphoreType.DMA],
    )
    _KERNELS["k3"] = pl.kernel(
        _k3_body,
        compiler_params=_CPARAMS,
        out_type=jax.ShapeDtypeStruct((B, N), _f32),
        mesh=mesh,
        scratch_types=[pltpu.VMEM((C,), _i32),
                       pltpu.VMEM((C, 16), _f32),
                       pltpu.VMEM((C, 16), _f32),
                       pltpu.VMEM((B, C), _f32),
                       pltpu.SemaphoreType.DMA],
    )
    return _KERNELS


def _bf16r(x):
    i = lax.bitcast_convert_type(x, jnp.int32)
    tie = jnp.bitwise_and(lax.shift_right_logical(i, 16), jnp.int32(1))
    i = i + jnp.int32(0x7FFF) + tie
    i = jnp.bitwise_and(i, jnp.int32(-65536))
    return lax.bitcast_convert_type(i, jnp.float32)


def _sum3_rne_jnp(p0, p1, p2):
    t = p0 + p1
    bv = t - p0
    e1 = (p0 - (t - bv)) + (p1 - bv)
    s = t + p2
    bv2 = s - t
    e2 = (t - (s - bv2)) + (p2 - bv2)
    return s + (e1 + e2)


def _mm3_bf16x1(A, Bm):
    Ab, Bb = _bf16r(A), _bf16r(Bm)
    p0 = Ab[:, :, 0, None] * Bb[:, 0, None, :].reshape(-1, 1, 3)
    p1 = Ab[:, :, 1, None] * Bb[:, 1, None, :].reshape(-1, 1, 3)
    p2 = Ab[:, :, 2, None] * Bb[:, 2, None, :].reshape(-1, 1, 3)
    return _sum3_rne_jnp(p0, p1, p2)


def kernel(geometry, euler, trans, cam, tris, vert_tris):
    Rx, Ry, Rz = _euler_rotmats(euler)
    R = _bf16r(_mm3_bf16x1(_mm3_bf16x1(Rz, Ry), Rx))
    params = jnp.concatenate([R.reshape(B, 9), trans, cam], axis=1)
    params_bc = jnp.broadcast_to(params[:, :, None], (B, 16, L)).astype(_f32)
    ks = _build_kernels()
    proj_geo, packed = ks["k1"](geometry, params_bc)
    rot_tri_normal, npacked = ks["k2"](tris, packed)
    is_visible = ks["k3"](vert_tris, npacked, packed)
    return (proj_geo, rot_tri_normal, is_visible)
